# Initial kernel scaffold; baseline (speedup 1.0000x reference)
import functools

import jax
import jax.numpy as jnp
from jax import lax
from jax.experimental import pallas as pl
from jax.experimental.pallas import tpu as pltpu
from jax.experimental.pallas import tpu_sc as plsc

N = 10000
D = 128
E = 320000
HEADS = 4
HC = 32

NC = 2
NS = 16
LANES = 128
ROWS = E // LANES
ROWS_PER_TILE = ROWS // NS
ROWS_EXTRA = ROWS - ROWS_PER_TILE * NS
ROWS_PER_W = ROWS // (NC * NS)
ROWS_W_EXTRA = ROWS - ROWS_PER_W * NC * NS
NPT = N // NS
NZ = 624
HD = D // NC

RB = 500
GRID = N // RB

_mesh = plsc.VectorSubcoreMesh(core_axis_name="c", subcore_axis_name="s")


def _lrelu(v):
    return jnp.where(v > 0, v, 0.2 * v)


@functools.partial(
    pl.kernel,
    out_type=jax.ShapeDtypeStruct((NC, N), jnp.float32),
    mesh=_mesh,
    scratch_types=[
        pltpu.VMEM_SHARED((N,), jnp.float32),
        pltpu.VMEM((1, LANES), jnp.int32),
        pltpu.VMEM((1, LANES), jnp.float32),
    ],
)
def _sc_deg(dst_hbm, ew_hbm, z_hbm, out_hbm, deg_sh, dstv, eww):
    cid = lax.axis_index("c")
    sid = lax.axis_index("s")
    z0 = sid * NZ
    pltpu.sync_copy(z_hbm.at[pl.ds(z0, NZ)], deg_sh.at[pl.ds(z0, NZ)])

    @pl.when(sid == NS - 1)
    def _():
        pltpu.sync_copy(z_hbm.at[pl.ds(NS * NZ, N - NS * NZ)],
                        deg_sh.at[pl.ds(NS * NZ, N - NS * NZ)])

    plsc.subcore_barrier()
    w = sid * NC + cid
    start = w * ROWS_PER_W + jnp.minimum(w, ROWS_W_EXTRA)
    cnt = ROWS_PER_W + jnp.where(w < ROWS_W_EXTRA, 1, 0)

    def row_body(i, carry):
        r = start + i
        pltpu.sync_copy(dst_hbm.at[r], dstv.at[0])
        pltpu.sync_copy(ew_hbm.at[r], eww.at[0])
        pltpu.sync_copy(eww.at[0], deg_sh.at[dstv.at[0]], add=True)
        return carry

    lax.fori_loop(0, cnt, row_body, 0)
    plsc.subcore_barrier()
    pltpu.sync_copy(deg_sh.at[pl.ds(z0, NZ)], out_hbm.at[cid, pl.ds(z0, NZ)])

    @pl.when(sid == NS - 1)
    def _():
        pltpu.sync_copy(deg_sh.at[pl.ds(NS * NZ, N - NS * NZ)],
                        out_hbm.at[cid, pl.ds(NS * NZ, N - NS * NZ)])


@functools.partial(
    pl.kernel,
    out_type=jax.ShapeDtypeStruct((NC, N, HD), jnp.float32),
    mesh=_mesh,
    scratch_types=[
        pltpu.VMEM_SHARED((N, HD), jnp.float32),
        pltpu.VMEM_SHARED((N, HD), jnp.float32),
        pltpu.VMEM((N, 2), jnp.float32),
        pltpu.VMEM((1, LANES), jnp.int32),
        pltpu.VMEM((1, LANES), jnp.int32),
        pltpu.VMEM((1, LANES), jnp.float32),
        pltpu.VMEM((LANES,), jnp.float32),
        pltpu.VMEM((LANES, HD), jnp.float32),
    ],
)
def _sc_gcn(h_hbm, src_hbm, dst_hbm, ew_hbm, dins_hbm, z_hbm, out_hbm,
            h_sh, acc_sh, dins_v, srcv, dstv, eww, nrmrow, rows_v):
    cid = lax.axis_index("c")
    sid = lax.axis_index("s")
    r0 = sid * NPT
    pltpu.sync_copy(h_hbm.at[cid, pl.ds(r0, NPT)], h_sh.at[pl.ds(r0, NPT)])
    pltpu.sync_copy(z_hbm.at[pl.ds(r0, NPT)], acc_sh.at[pl.ds(r0, NPT)])
    pltpu.sync_copy(dins_hbm, dins_v)
    plsc.subcore_barrier()

    lanes = lax.iota(jnp.int32, 16)
    zcol = lanes * 0
    start = sid * ROWS_PER_TILE + jnp.minimum(sid, ROWS_EXTRA)
    cnt = ROWS_PER_TILE + jnp.where(sid < ROWS_EXTRA, 1, 0)

    def row_body(i, carry):
        r = start + i
        pltpu.sync_copy(src_hbm.at[r], srcv.at[0])
        pltpu.sync_copy(dst_hbm.at[r], dstv.at[0])
        pltpu.sync_copy(ew_hbm.at[r], eww.at[0])
        pltpu.sync_copy(h_sh.at[srcv.at[0]], rows_v)

        def grp(g, c2):
            s16 = srcv[0, pl.ds(g * 16, 16)]
            d16 = dstv[0, pl.ds(g * 16, 16)]
            w16 = eww[0, pl.ds(g * 16, 16)]
            dis = plsc.load_gather(dins_v, [s16, zcol])
            did = plsc.load_gather(dins_v, [d16, zcol])
            nrmrow[pl.ds(g * 16, 16)] = dis * w16 * did
            return c2

        lax.fori_loop(0, LANES // 16, grp, 0)

        def edge(e, c2):
            nr = plsc.load_gather(nrmrow, [jnp.broadcast_to(e, (16,))])
            for j in range(HD // 16):
                sl = pl.ds(j * 16, 16)
                rows_v[e, sl] = rows_v[e, sl] * nr
            return c2

        lax.fori_loop(0, LANES, edge, 0)
        pltpu.sync_copy(rows_v, acc_sh.at[dstv.at[0]], add=True)
        return carry

    lax.fori_loop(0, cnt, row_body, 0)
    plsc.subcore_barrier()
    pltpu.sync_copy(acc_sh.at[pl.ds(r0, NPT)], out_hbm.at[cid, pl.ds(r0, NPT)])


@functools.partial(
    pl.kernel,
    out_type=[
        jax.ShapeDtypeStruct((NC, N, HD), jnp.float32),
        jax.ShapeDtypeStruct((NC, N, 2), jnp.float32),
    ],
    mesh=_mesh,
    scratch_types=[
        pltpu.VMEM_SHARED((N, HD), jnp.float32),
        pltpu.VMEM_SHARED((N, HD), jnp.float32),
        pltpu.VMEM_SHARED((N, 2), jnp.float32),
        pltpu.VMEM((N, HEADS), jnp.float32),
        pltpu.VMEM((N, HEADS), jnp.float32),
        pltpu.VMEM((LANES,), jnp.float32),
        pltpu.VMEM((1, LANES), jnp.int32),
        pltpu.VMEM((1, LANES), jnp.int32),
        pltpu.VMEM((LANES, 2), jnp.float32),
        pltpu.VMEM((LANES, HD), jnp.float32),
    ],
)
def _sc_gat(h_hbm, src_hbm, dst_hbm, as_hbm, ad_hbm, gm_hbm, z_hbm, z2_hbm,
            agg_hbm, den_hbm,
            h_sh, acc_sh, den_sh, as_v, ad_v, gmx, srcv, dstv, exrow, rows_v):
    cid = lax.axis_index("c")
    sid = lax.axis_index("s")
    r0 = sid * NPT
    z0 = sid * NZ
    pltpu.sync_copy(h_hbm.at[cid, pl.ds(r0, NPT)], h_sh.at[pl.ds(r0, NPT)])
    pltpu.sync_copy(z_hbm.at[pl.ds(r0, NPT)], acc_sh.at[pl.ds(r0, NPT)])
    pltpu.sync_copy(z2_hbm.at[pl.ds(z0, NZ)], den_sh.at[pl.ds(z0, NZ)])

    @pl.when(sid == NS - 1)
    def _():
        pltpu.sync_copy(z2_hbm.at[pl.ds(NS * NZ, N - NS * NZ)],
                        den_sh.at[pl.ds(NS * NZ, N - NS * NZ)])

    pltpu.sync_copy(as_hbm, as_v)
    pltpu.sync_copy(ad_hbm, ad_v)
    pltpu.sync_copy(gm_hbm.at[0], gmx)
    plsc.subcore_barrier()

    lanes = lax.iota(jnp.int32, 16)
    zcol = lanes * 0
    col0 = (zcol + 2 * cid).astype(jnp.int32)
    col1 = col0 + 1
    gm0 = plsc.load_gather(gmx, [col0])
    gm1 = plsc.load_gather(gmx, [col1])
    start = sid * ROWS_PER_TILE + jnp.minimum(sid, ROWS_EXTRA)
    cnt = ROWS_PER_TILE + jnp.where(sid < ROWS_EXTRA, 1, 0)

    def row_body(i, carry):
        r = start + i
        pltpu.sync_copy(src_hbm.at[r], srcv.at[0])
        pltpu.sync_copy(dst_hbm.at[r], dstv.at[0])
        pltpu.sync_copy(h_sh.at[srcv.at[0]], rows_v)

        def grp(g, c2):
            s16 = srcv[0, pl.ds(g * 16, 16)]
            d16 = dstv[0, pl.ds(g * 16, 16)]
            as0 = plsc.load_gather(as_v, [s16, col0])
            as1 = plsc.load_gather(as_v, [s16, col1])
            ad0 = plsc.load_gather(ad_v, [d16, col0])
            ad1 = plsc.load_gather(ad_v, [d16, col1])
            ex0 = jnp.exp(_lrelu(as0 + ad0) - _lrelu(gm0 + ad0))
            ex1 = jnp.exp(_lrelu(as1 + ad1) - _lrelu(gm1 + ad1))
            idx16 = lanes + g * 16
            plsc.store_scatter(exrow, [idx16, zcol], ex0)
            plsc.store_scatter(exrow, [idx16, zcol + 1], ex1)
            return c2

        lax.fori_loop(0, LANES // 16, grp, 0)
        pltpu.sync_copy(exrow, den_sh.at[dstv.at[0]], add=True)

        def edge(e, c2):
            e16 = jnp.broadcast_to(e, (16,))
            x0 = plsc.load_gather(exrow, [e16, zcol])
            x1 = plsc.load_gather(exrow, [e16, zcol + 1])
            rows_v[e, pl.ds(0, 16)] = rows_v[e, pl.ds(0, 16)] * x0
            rows_v[e, pl.ds(16, 16)] = rows_v[e, pl.ds(16, 16)] * x0
            rows_v[e, pl.ds(32, 16)] = rows_v[e, pl.ds(32, 16)] * x1
            rows_v[e, pl.ds(48, 16)] = rows_v[e, pl.ds(48, 16)] * x1
            return c2

        lax.fori_loop(0, LANES, edge, 0)
        pltpu.sync_copy(rows_v, acc_sh.at[dstv.at[0]], add=True)
        return carry

    lax.fori_loop(0, cnt, row_body, 0)
    plsc.subcore_barrier()
    pltpu.sync_copy(acc_sh.at[pl.ds(r0, NPT)], agg_hbm.at[cid, pl.ds(r0, NPT)])
    pltpu.sync_copy(den_sh.at[pl.ds(z0, NZ)], den_hbm.at[cid, pl.ds(z0, NZ)])

    @pl.when(sid == NS - 1)
    def _():
        pltpu.sync_copy(den_sh.at[pl.ds(NS * NZ, N - NS * NZ)],
                        den_hbm.at[cid, pl.ds(NS * NZ, N - NS * NZ)])


def _tc1_body(x_ref, w_ref, dg_ref, h_ref, dins_ref):
    h = jnp.dot(x_ref[...], w_ref[...], preferred_element_type=jnp.float32)
    h_ref[0] = h[:, :HD]
    h_ref[1] = h[:, HD:]
    deg = 2.0 + dg_ref[0] + dg_ref[1]
    dinv = jnp.where(deg > 0, lax.rsqrt(jnp.maximum(deg, 1e-12)), 0.0)
    dins_ref[...] = jnp.stack([dinv, 2.0 * dinv * dinv], axis=-1)


_tc_prep1 = pl.pallas_call(
    _tc1_body,
    grid=(GRID,),
    in_specs=[
        pl.BlockSpec((RB, D), lambda i: (i, 0)),
        pl.BlockSpec((D, D), lambda i: (0, 0)),
        pl.BlockSpec((NC, RB), lambda i: (0, i)),
    ],
    out_specs=[
        pl.BlockSpec((NC, RB, HD), lambda i: (0, i, 0)),
        pl.BlockSpec((RB, 2), lambda i: (i, 0)),
    ],
    out_shape=[
        jax.ShapeDtypeStruct((NC, N, HD), jnp.float32),
        jax.ShapeDtypeStruct((N, 2), jnp.float32),
    ],
)


def _tc2_body(x_ref, agg_ref, h0_ref, dins_ref, b0_ref, wg_ref, as_ref,
              ad_ref, xp_ref, h1_ref, as4_ref, ad4_ref, gm_ref):
    s = dins_ref[:, 1:2]
    xa0 = jnp.maximum(agg_ref[0] + s * h0_ref[0] + b0_ref[0, :HD], 0.0)
    xa1 = jnp.maximum(agg_ref[1] + s * h0_ref[1] + b0_ref[0, HD:], 0.0)
    xp_ref[0] = xa0
    xp_ref[1] = xa1
    xs = x_ref[...] + jnp.concatenate([xa0, xa1], axis=1)
    h1 = jnp.dot(xs, wg_ref[...], preferred_element_type=jnp.float32)
    h1_ref[0] = h1[:, :HD]
    h1_ref[1] = h1[:, HD:]
    a_s = jnp.dot(h1, as_ref[...], preferred_element_type=jnp.float32)
    a_d = jnp.dot(h1, ad_ref[...], preferred_element_type=jnp.float32)
    as4_ref[...] = a_s
    ad4_ref[...] = a_d
    bm = jnp.max(a_s, axis=0)
    cur = jnp.broadcast_to(
        jnp.pad(bm, (0, D - HEADS), constant_values=-1e30)[None, :], (8, D))
    i = pl.program_id(0)

    @pl.when(i == 0)
    def _():
        gm_ref[...] = cur

    @pl.when(i > 0)
    def _():
        gm_ref[...] = jnp.maximum(gm_ref[...], cur)


_tc_prep2 = pl.pallas_call(
    _tc2_body,
    grid=(GRID,),
    in_specs=[
        pl.BlockSpec((RB, D), lambda i: (i, 0)),
        pl.BlockSpec((NC, RB, HD), lambda i: (0, i, 0)),
        pl.BlockSpec((NC, RB, HD), lambda i: (0, i, 0)),
        pl.BlockSpec((RB, 2), lambda i: (i, 0)),
        pl.BlockSpec((1, D), lambda i: (0, 0)),
        pl.BlockSpec((D, D), lambda i: (0, 0)),
        pl.BlockSpec((D, HEADS), lambda i: (0, 0)),
        pl.BlockSpec((D, HEADS), lambda i: (0, 0)),
    ],
    out_specs=[
        pl.BlockSpec((NC, RB, HD), lambda i: (0, i, 0)),
        pl.BlockSpec((NC, RB, HD), lambda i: (0, i, 0)),
        pl.BlockSpec((RB, HEADS), lambda i: (i, 0)),
        pl.BlockSpec((RB, HEADS), lambda i: (i, 0)),
        pl.BlockSpec((8, D), lambda i: (0, 0)),
    ],
    out_shape=[
        jax.ShapeDtypeStruct((NC, N, HD), jnp.float32),
        jax.ShapeDtypeStruct((NC, N, HD), jnp.float32),
        jax.ShapeDtypeStruct((N, HEADS), jnp.float32),
        jax.ShapeDtypeStruct((N, HEADS), jnp.float32),
        jax.ShapeDtypeStruct((8, D), jnp.float32),
    ],
)


def _tc3_body(agg_ref, den_ref, as4_ref, ad4_ref, gm_ref, h1_ref, xp_ref,
              bg_ref, wl_ref, h2_ref):
    gm = gm_ref[0:1, :HEADS]
    ad = ad4_ref[...]
    m = _lrelu(gm + ad)
    exs = jnp.exp(_lrelu(as4_ref[...] + ad) - m)
    dent = jnp.concatenate([den_ref[0], den_ref[1]], axis=1) + exs + 1e-16
    halves = []
    for c in range(NC):
        e0 = exs[:, 2 * c:2 * c + 1]
        e1 = exs[:, 2 * c + 1:2 * c + 2]
        h1c = h1_ref[c]
        num = agg_ref[c] + jnp.concatenate(
            [e0 * h1c[:, :HC], e1 * h1c[:, HC:]], axis=1)
        d0 = jnp.broadcast_to(dent[:, 2 * c:2 * c + 1], (RB, HC))
        d1 = jnp.broadcast_to(dent[:, 2 * c + 1:2 * c + 2], (RB, HC))
        den64 = jnp.concatenate([d0, d1], axis=1)
        xac = jnp.maximum(num / den64 + bg_ref[0, HD * c:HD * c + HD], 0.0)
        halves.append(xp_ref[c] + xac)
    xs2 = jnp.concatenate(halves, axis=1)
    h2 = jnp.dot(xs2, wl_ref[...], preferred_element_type=jnp.float32)
    h2_ref[0] = h2[:, :HD]
    h2_ref[1] = h2[:, HD:]


_tc_epi2 = pl.pallas_call(
    _tc3_body,
    grid=(GRID,),
    in_specs=[
        pl.BlockSpec((NC, RB, HD), lambda i: (0, i, 0)),
        pl.BlockSpec((NC, RB, 2), lambda i: (0, i, 0)),
        pl.BlockSpec((RB, HEADS), lambda i: (i, 0)),
        pl.BlockSpec((RB, HEADS), lambda i: (i, 0)),
        pl.BlockSpec((8, D), lambda i: (0, 0)),
        pl.BlockSpec((NC, RB, HD), lambda i: (0, i, 0)),
        pl.BlockSpec((NC, RB, HD), lambda i: (0, i, 0)),
        pl.BlockSpec((1, D), lambda i: (0, 0)),
        pl.BlockSpec((D, D), lambda i: (0, 0)),
    ],
    out_specs=[
        pl.BlockSpec((NC, RB, HD), lambda i: (0, i, 0)),
    ],
    out_shape=[
        jax.ShapeDtypeStruct((NC, N, HD), jnp.float32),
    ],
)


def _tc4_body(agg_ref, h2_ref, dins_ref, bl_ref, out_ref):
    s = dins_ref[:, 1:2]
    o0 = agg_ref[0] + s * h2_ref[0] + bl_ref[0, :HD]
    o1 = agg_ref[1] + s * h2_ref[1] + bl_ref[0, HD:]
    out_ref[...] = jnp.concatenate([o0, o1], axis=1)


_tc_final = pl.pallas_call(
    _tc4_body,
    grid=(GRID,),
    in_specs=[
        pl.BlockSpec((NC, RB, HD), lambda i: (0, i, 0)),
        pl.BlockSpec((NC, RB, HD), lambda i: (0, i, 0)),
        pl.BlockSpec((RB, 2), lambda i: (i, 0)),
        pl.BlockSpec((1, D), lambda i: (0, 0)),
    ],
    out_specs=pl.BlockSpec((RB, D), lambda i: (i, 0)),
    out_shape=jax.ShapeDtypeStruct((N, D), jnp.float32),
)


def kernel(x, edge_index, edge_weight, W0, b0, Wg0, att_src0, att_dst0,
           bg0, Wl, bl):
    src2d = edge_index[0].reshape(ROWS, LANES)
    dst2d = edge_index[1].reshape(ROWS, LANES)
    ew2d = edge_weight.reshape(ROWS, LANES)
    zeros64 = jnp.zeros((N, HD), jnp.float32)
    zcol = jnp.zeros((N,), jnp.float32)
    zeros2 = jnp.zeros((N, 2), jnp.float32)
    b0r = b0.reshape(1, D)
    bgr = bg0.reshape(1, D)
    blr = bl.reshape(1, D)
    As = jnp.zeros((D, HEADS), jnp.float32)
    Ad = jnp.zeros((D, HEADS), jnp.float32)
    for h in range(HEADS):
        As = As.at[h * HC:(h + 1) * HC, h].set(att_src0[0, h, :])
        Ad = Ad.at[h * HC:(h + 1) * HC, h].set(att_dst0[0, h, :])

    degp = _sc_deg(dst2d, ew2d, zcol)
    h0, dins = _tc_prep1(x, W0, degp)
    agg1 = _sc_gcn(h0, src2d, dst2d, ew2d, dins, zeros64)
    xp, h1, as4, ad4, gmax = _tc_prep2(x, agg1, h0, dins, b0r, Wg0, As, Ad)
    agg2, den = _sc_gat(h1, src2d, dst2d, as4, ad4, gmax, zeros64, zeros2)
    h2, = _tc_epi2(agg2, den, as4, ad4, gmax, h1, xp, bgr, Wl)
    agg3 = _sc_gcn(h2, src2d, dst2d, ew2d, dins, zeros64)
    out = _tc_final(agg3, h2, dins, blr)
    return out


# speedup vs baseline: 37.1233x; 37.1233x over previous
import functools

import jax
import jax.numpy as jnp
from jax import lax
from jax.experimental import pallas as pl
from jax.experimental.pallas import tpu as pltpu
from jax.experimental.pallas import tpu_sc as plsc

N = 10000
D = 128
E = 320000
HEADS = 4
HC = 32

NC = 2
NS = 16
LANES = 128
ROWS = E // LANES
BSC = ROWS // NC
BPT = BSC // NS
BEX = BSC - BPT * NS
ROWS_PER_W = ROWS // (NC * NS)
ROWS_W_EXTRA = ROWS - ROWS_PER_W * NC * NS
NZ = 624
NTAIL = N - NS * NZ
NZH = 2496
HTAIL = HEADS * N - NS * NZH

RB = 1000
GRID = N // RB

_mesh = plsc.VectorSubcoreMesh(core_axis_name="c", subcore_axis_name="s")


def _hops(total, step=128):
    out = []
    o = 0
    while o < total:
        s = min(step, total - o)
        out.append((o, s))
        o += s
    return out


def _lrelu(v):
    return jnp.where(v > 0, v, 0.2 * v)


_GDN = lax.GatherDimensionNumbers(
    offset_dims=(), collapsed_slice_dims=(0,), start_index_map=(0,))


def _splat(v16, j):
    idx = jnp.full((16, 1), j, jnp.int32)
    return lax.gather(v16, idx, _GDN, (1,),
                      mode=lax.GatherScatterMode.PROMISE_IN_BOUNDS)


@functools.partial(
    pl.kernel,
    out_type=jax.ShapeDtypeStruct((NC * N,), jnp.float32),
    mesh=_mesh,
    scratch_types=[
        pltpu.VMEM_SHARED((N,), jnp.float32),
        pltpu.VMEM((1, LANES), jnp.int32),
        pltpu.VMEM((1, LANES), jnp.float32),
    ],
)
def _sc_deg(dst_hbm, ew_hbm, out_hbm, deg_sh, dstv, eww):
    cid = lax.axis_index("c")
    sid = lax.axis_index("s")
    z0 = sid * NZ
    for j in range(LANES // 16):
        eww[0, pl.ds(j * 16, 16)] = jnp.zeros((16,), jnp.float32)
    for o, s in _hops(NZ):
        pltpu.sync_copy(eww.at[0, pl.ds(0, s)], deg_sh.at[pl.ds(z0 + o, s)])

    @pl.when(sid == NS - 1)
    def _():
        pltpu.sync_copy(eww.at[0, pl.ds(0, NTAIL)],
                        deg_sh.at[pl.ds(NS * NZ, NTAIL)])

    plsc.subcore_barrier()
    w = sid * NC + cid
    start = w * ROWS_PER_W + jnp.minimum(w, ROWS_W_EXTRA)
    cnt = ROWS_PER_W + jnp.where(w < ROWS_W_EXTRA, 1, 0)

    def row_body(i, carry):
        r = start + i
        pltpu.sync_copy(dst_hbm.at[pl.ds(r * LANES, LANES)], dstv.at[0])
        pltpu.sync_copy(ew_hbm.at[pl.ds(r * LANES, LANES)], eww.at[0])
        pltpu.sync_copy(eww.at[0], deg_sh.at[dstv.at[0]], add=True)
        return carry

    lax.fori_loop(0, cnt, row_body, 0)
    plsc.subcore_barrier()
    for o, s in _hops(NZ):
        pltpu.sync_copy(deg_sh.at[pl.ds(z0 + o, s)], eww.at[0, pl.ds(0, s)])
        pltpu.sync_copy(eww.at[0, pl.ds(0, s)],
                        out_hbm.at[pl.ds(cid * N + z0 + o, s)])

    @pl.when(sid == NS - 1)
    def _():
        pltpu.sync_copy(deg_sh.at[pl.ds(NS * NZ, NTAIL)],
                        eww.at[0, pl.ds(0, NTAIL)])
        pltpu.sync_copy(eww.at[0, pl.ds(0, NTAIL)],
                        out_hbm.at[pl.ds(cid * N + NS * NZ, NTAIL)])


@functools.partial(
    pl.kernel,
    out_type=jax.ShapeDtypeStruct((NC, N, D), jnp.float32),
    mesh=_mesh,
    scratch_types=[
        pltpu.VMEM_SHARED((N, D), jnp.float32),
        pltpu.VMEM((1, LANES), jnp.int32),
        pltpu.VMEM((1, LANES), jnp.int32),
        pltpu.VMEM((1, LANES), jnp.float32),
        pltpu.VMEM((LANES, D), jnp.float32),
    ],
)
def _sc_gcn(h_hbm, src_hbm, dst_hbm, ew_hbm, out_hbm,
            acc_sh, srcv, dstv, eww, rows_v):
    cid = lax.axis_index("c")
    sid = lax.axis_index("s")
    row0 = sid * NZ
    last = sid == NS - 1
    z16 = jnp.zeros((16,), jnp.float32)

    for e in range(LANES):
        for j in range(D // 16):
            rows_v[e, pl.ds(j * 16, 16)] = z16
    for o, s in _hops(NZ):
        pltpu.sync_copy(rows_v.at[pl.ds(0, s)], acc_sh.at[pl.ds(row0 + o, s)])

    @pl.when(last)
    def _():
        pltpu.sync_copy(rows_v.at[pl.ds(0, NTAIL)],
                        acc_sh.at[pl.ds(NS * NZ, NTAIL)])

    plsc.subcore_barrier()

    start = cid * BSC + sid * BPT + jnp.minimum(sid, BEX)
    cnt = BPT + jnp.where(sid < BEX, 1, 0)

    def row_body(i, carry):
        r = start + i
        pltpu.sync_copy(src_hbm.at[pl.ds(r * LANES, LANES)], srcv.at[0])
        pltpu.sync_copy(dst_hbm.at[pl.ds(r * LANES, LANES)], dstv.at[0])
        pltpu.sync_copy(ew_hbm.at[pl.ds(r * LANES, LANES)], eww.at[0])
        pltpu.sync_copy(h_hbm.at[srcv.at[0]], rows_v)

        for g in range(LANES // 16):
            w16 = eww[0, pl.ds(g * 16, 16)]
            for e in range(16):
                b = _splat(w16, e)
                row = g * 16 + e
                for j in range(D // 16):
                    sl = pl.ds(j * 16, 16)
                    rows_v[row, sl] = rows_v[row, sl] * b

        pltpu.sync_copy(rows_v, acc_sh.at[dstv.at[0]], add=True)
        return carry

    lax.fori_loop(0, cnt, row_body, 0)
    plsc.subcore_barrier()
    for o, s in _hops(NZ):
        pltpu.sync_copy(acc_sh.at[pl.ds(row0 + o, s)], rows_v.at[pl.ds(0, s)])
        pltpu.sync_copy(rows_v.at[pl.ds(0, s)],
                        out_hbm.at[cid, pl.ds(row0 + o, s)])

    @pl.when(last)
    def _():
        pltpu.sync_copy(acc_sh.at[pl.ds(NS * NZ, NTAIL)],
                        rows_v.at[pl.ds(0, NTAIL)])
        pltpu.sync_copy(rows_v.at[pl.ds(0, NTAIL)],
                        out_hbm.at[cid, pl.ds(NS * NZ, NTAIL)])


@functools.partial(
    pl.kernel,
    out_type=[
        jax.ShapeDtypeStruct((NC, N, D), jnp.float32),
        jax.ShapeDtypeStruct((NC * HEADS * N,), jnp.float32),
    ],
    mesh=_mesh,
    scratch_types=[
        pltpu.VMEM_SHARED((N, D), jnp.float32),
        pltpu.VMEM_SHARED((N,), jnp.float32),
        pltpu.VMEM_SHARED((N,), jnp.float32),
        pltpu.VMEM_SHARED((N,), jnp.float32),
        pltpu.VMEM_SHARED((N,), jnp.float32),
        pltpu.VMEM_SHARED((HEADS * N,), jnp.float32),
        pltpu.VMEM_SHARED((HEADS * N,), jnp.float32),
        pltpu.VMEM((1, LANES), jnp.int32),
        pltpu.VMEM((1, LANES), jnp.int32),
        pltpu.VMEM((1, LANES), jnp.int32),
        pltpu.VMEM((1, LANES), jnp.float32),
        pltpu.VMEM((1, LANES), jnp.float32),
        pltpu.VMEM((1, LANES), jnp.float32),
        pltpu.VMEM((1, LANES), jnp.float32),
        pltpu.VMEM((1, LANES), jnp.float32),
        pltpu.VMEM((1, LANES), jnp.float32),
        pltpu.VMEM((1, 16), jnp.float32),
        pltpu.VMEM((LANES, D), jnp.float32),
    ],
)
def _sc_gat(h_hbm, src_hbm, dst_hbm, as_hbm, ad_hbm, gm_hbm,
            agg_hbm, den_hbm,
            acc_sh, den0_sh, den1_sh, den2_sh, den3_sh, as_sh, ad_sh,
            srcv, dstv, idxb, asr, adr, ex0r, ex1r, ex2r, ex3r, gmv,
            rows_v):
    cid = lax.axis_index("c")
    sid = lax.axis_index("s")
    row0 = sid * NZ
    last = sid == NS - 1
    z16 = jnp.zeros((16,), jnp.float32)
    dens = [den0_sh, den1_sh, den2_sh, den3_sh]
    exrs = [ex0r, ex1r, ex2r, ex3r]

    for e in range(LANES):
        for j in range(D // 16):
            rows_v[e, pl.ds(j * 16, 16)] = z16
    for j in range(LANES // 16):
        ex0r[0, pl.ds(j * 16, 16)] = z16
    for o, s in _hops(NZ):
        pltpu.sync_copy(rows_v.at[pl.ds(0, s)], acc_sh.at[pl.ds(row0 + o, s)])
        for dh in dens:
            pltpu.sync_copy(ex0r.at[0, pl.ds(0, s)],
                            dh.at[pl.ds(row0 + o, s)])

    @pl.when(last)
    def _():
        pltpu.sync_copy(rows_v.at[pl.ds(0, NTAIL)],
                        acc_sh.at[pl.ds(NS * NZ, NTAIL)])
        for dh in dens:
            pltpu.sync_copy(ex0r.at[0, pl.ds(0, NTAIL)],
                            dh.at[pl.ds(NS * NZ, NTAIL)])

    h0 = sid * NZH
    for o, s in _hops(NZH):
        pltpu.sync_copy(as_hbm.at[pl.ds(h0 + o, s)], ex1r.at[0, pl.ds(0, s)])
        pltpu.sync_copy(ex1r.at[0, pl.ds(0, s)], as_sh.at[pl.ds(h0 + o, s)])
        pltpu.sync_copy(ad_hbm.at[pl.ds(h0 + o, s)], ex1r.at[0, pl.ds(0, s)])
        pltpu.sync_copy(ex1r.at[0, pl.ds(0, s)], ad_sh.at[pl.ds(h0 + o, s)])

    @pl.when(last)
    def _():
        pltpu.sync_copy(as_hbm.at[pl.ds(NS * NZH, HTAIL)],
                        ex1r.at[0, pl.ds(0, HTAIL)])
        pltpu.sync_copy(ex1r.at[0, pl.ds(0, HTAIL)],
                        as_sh.at[pl.ds(NS * NZH, HTAIL)])
        pltpu.sync_copy(ad_hbm.at[pl.ds(NS * NZH, HTAIL)],
                        ex1r.at[0, pl.ds(0, HTAIL)])
        pltpu.sync_copy(ex1r.at[0, pl.ds(0, HTAIL)],
                        ad_sh.at[pl.ds(NS * NZH, HTAIL)])

    pltpu.sync_copy(gm_hbm.at[pl.ds(0, 16)], gmv.at[0])
    plsc.subcore_barrier()

    gms = [_splat(gmv[0, pl.ds(0, 16)], h) for h in range(HEADS)]

    start = cid * BSC + sid * BPT + jnp.minimum(sid, BEX)
    cnt = BPT + jnp.where(sid < BEX, 1, 0)

    def row_body(i, carry):
        r = start + i
        pltpu.sync_copy(src_hbm.at[pl.ds(r * LANES, LANES)], srcv.at[0])
        pltpu.sync_copy(dst_hbm.at[pl.ds(r * LANES, LANES)], dstv.at[0])
        pltpu.sync_copy(h_hbm.at[srcv.at[0]], rows_v)

        for h in range(HEADS):
            for g in range(LANES // 16):
                sl = pl.ds(g * 16, 16)
                idxb[0, sl] = lax.shift_left(srcv[0, sl], 2) + h
            pltpu.sync_copy(as_sh.at[idxb.at[0]], asr.at[0])
            for g in range(LANES // 16):
                sl = pl.ds(g * 16, 16)
                idxb[0, sl] = lax.shift_left(dstv[0, sl], 2) + h
            pltpu.sync_copy(ad_sh.at[idxb.at[0]], adr.at[0])
            for g in range(LANES // 16):
                sl = pl.ds(g * 16, 16)
                ad16 = adr[0, sl]
                exrs[h][0, sl] = jnp.exp(_lrelu(asr[0, sl] + ad16)
                                         - _lrelu(gms[h] + ad16))
            pltpu.sync_copy(exrs[h].at[0], dens[h].at[dstv.at[0]], add=True)

        for g in range(LANES // 16):
            sl16 = pl.ds(g * 16, 16)
            x16 = [exrs[h][0, sl16] for h in range(HEADS)]
            for e in range(16):
                row = g * 16 + e
                for h in range(HEADS):
                    b = _splat(x16[h], e)
                    sl_a = pl.ds(h * HC, 16)
                    sl_b = pl.ds(h * HC + 16, 16)
                    rows_v[row, sl_a] = rows_v[row, sl_a] * b
                    rows_v[row, sl_b] = rows_v[row, sl_b] * b

        pltpu.sync_copy(rows_v, acc_sh.at[dstv.at[0]], add=True)
        return carry

    lax.fori_loop(0, cnt, row_body, 0)
    plsc.subcore_barrier()
    for o, s in _hops(NZ):
        pltpu.sync_copy(acc_sh.at[pl.ds(row0 + o, s)], rows_v.at[pl.ds(0, s)])
        pltpu.sync_copy(rows_v.at[pl.ds(0, s)],
                        agg_hbm.at[cid, pl.ds(row0 + o, s)])
        for h in range(HEADS):
            pltpu.sync_copy(dens[h].at[pl.ds(row0 + o, s)],
                            ex0r.at[0, pl.ds(0, s)])
            pltpu.sync_copy(
                ex0r.at[0, pl.ds(0, s)],
                den_hbm.at[pl.ds((cid * HEADS + h) * N + row0 + o, s)])

    @pl.when(last)
    def _():
        pltpu.sync_copy(acc_sh.at[pl.ds(NS * NZ, NTAIL)],
                        rows_v.at[pl.ds(0, NTAIL)])
        pltpu.sync_copy(rows_v.at[pl.ds(0, NTAIL)],
                        agg_hbm.at[cid, pl.ds(NS * NZ, NTAIL)])
        for h in range(HEADS):
            pltpu.sync_copy(dens[h].at[pl.ds(NS * NZ, NTAIL)],
                            ex0r.at[0, pl.ds(0, NTAIL)])
            pltpu.sync_copy(
                ex0r.at[0, pl.ds(0, NTAIL)],
                den_hbm.at[pl.ds((cid * HEADS + h) * N + NS * NZ, NTAIL)])


def _tc1_body(x_ref, w_ref, dg_ref, h_ref, hd_ref, dins_ref):
    h = jnp.dot(x_ref[...], w_ref[...], preferred_element_type=jnp.float32)
    deg = 2.0 + dg_ref[:, 0:1] + dg_ref[:, 1:2]
    dinv = jnp.where(deg > 0, lax.rsqrt(jnp.maximum(deg, 1e-12)), 0.0)
    dins_ref[...] = jnp.concatenate([dinv, 2.0 * dinv * dinv], axis=1)
    h_ref[...] = h
    hd_ref[...] = h * dinv


_tc_prep1 = pl.pallas_call(
    _tc1_body,
    grid=(GRID,),
    in_specs=[
        pl.BlockSpec((RB, D), lambda i: (i, 0)),
        pl.BlockSpec((D, D), lambda i: (0, 0)),
        pl.BlockSpec((RB, NC), lambda i: (i, 0)),
    ],
    out_specs=[
        pl.BlockSpec((RB, D), lambda i: (i, 0)),
        pl.BlockSpec((RB, D), lambda i: (i, 0)),
        pl.BlockSpec((RB, 2), lambda i: (i, 0)),
    ],
    out_shape=[
        jax.ShapeDtypeStruct((N, D), jnp.float32),
        jax.ShapeDtypeStruct((N, D), jnp.float32),
        jax.ShapeDtypeStruct((N, 2), jnp.float32),
    ],
)


def _tc2_body(x_ref, agg_ref, h0_ref, dins_ref, b0_ref, wg_ref, as_ref,
              ad_ref, xp_ref, h1_ref, as4_ref, ad4_ref, gm_ref):
    di = dins_ref[:, 0:1]
    s = dins_ref[:, 1:2]
    agg = agg_ref[0] + agg_ref[1]
    xa = jnp.maximum(di * agg + s * h0_ref[...] + b0_ref[0, :], 0.0)
    xp_ref[...] = xa
    xs = x_ref[...] + xa
    h1 = jnp.dot(xs, wg_ref[...], preferred_element_type=jnp.float32)
    h1_ref[...] = h1
    a_s = jnp.dot(h1, as_ref[...], preferred_element_type=jnp.float32)
    a_d = jnp.dot(h1, ad_ref[...], preferred_element_type=jnp.float32)
    as4_ref[...] = a_s
    ad4_ref[...] = a_d
    bm = jnp.max(a_s, axis=0)
    cur = jnp.broadcast_to(
        jnp.pad(bm, (0, D - HEADS), constant_values=-1e30)[None, :], (8, D))
    i = pl.program_id(0)

    @pl.when(i == 0)
    def _():
        gm_ref[...] = cur

    @pl.when(i > 0)
    def _():
        gm_ref[...] = jnp.maximum(gm_ref[...], cur)


_tc_prep2 = pl.pallas_call(
    _tc2_body,
    grid=(GRID,),
    in_specs=[
        pl.BlockSpec((RB, D), lambda i: (i, 0)),
        pl.BlockSpec((NC, RB, D), lambda i: (0, i, 0)),
        pl.BlockSpec((RB, D), lambda i: (i, 0)),
        pl.BlockSpec((RB, 2), lambda i: (i, 0)),
        pl.BlockSpec((1, D), lambda i: (0, 0)),
        pl.BlockSpec((D, D), lambda i: (0, 0)),
        pl.BlockSpec((D, HEADS), lambda i: (0, 0)),
        pl.BlockSpec((D, HEADS), lambda i: (0, 0)),
    ],
    out_specs=[
        pl.BlockSpec((RB, D), lambda i: (i, 0)),
        pl.BlockSpec((RB, D), lambda i: (i, 0)),
        pl.BlockSpec((RB, HEADS), lambda i: (i, 0)),
        pl.BlockSpec((RB, HEADS), lambda i: (i, 0)),
        pl.BlockSpec((8, D), lambda i: (0, 0)),
    ],
    out_shape=[
        jax.ShapeDtypeStruct((N, D), jnp.float32),
        jax.ShapeDtypeStruct((N, D), jnp.float32),
        jax.ShapeDtypeStruct((N, HEADS), jnp.float32),
        jax.ShapeDtypeStruct((N, HEADS), jnp.float32),
        jax.ShapeDtypeStruct((8, D), jnp.float32),
    ],
)


def _tc3_body(agg_ref, den_ref, as4_ref, ad4_ref, gm_ref, h1_ref, xp_ref,
              dins_ref, bg_ref, wl_ref, h2_ref, h2d_ref):
    gm = gm_ref[0:1, :HEADS]
    ad = ad4_ref[...]
    m = _lrelu(gm + ad)
    exs = jnp.exp(_lrelu(as4_ref[...] + ad) - m)
    dent = den_ref[0] + den_ref[1] + exs + 1e-16
    agg = agg_ref[0] + agg_ref[1]
    h1 = h1_ref[...]
    num_parts = []
    den_parts = []
    for h in range(HEADS):
        eh = exs[:, h:h + 1]
        num_parts.append(eh * h1[:, h * HC:(h + 1) * HC])
        den_parts.append(jnp.broadcast_to(dent[:, h:h + 1], (RB, HC)))
    num = agg + jnp.concatenate(num_parts, axis=1)
    den128 = jnp.concatenate(den_parts, axis=1)
    xa2 = jnp.maximum(num / den128 + bg_ref[0, :], 0.0)
    xs2 = xp_ref[...] + xa2
    h2 = jnp.dot(xs2, wl_ref[...], preferred_element_type=jnp.float32)
    di = dins_ref[:, 0:1]
    h2_ref[...] = h2
    h2d_ref[...] = h2 * di


_tc_epi2 = pl.pallas_call(
    _tc3_body,
    grid=(GRID,),
    in_specs=[
        pl.BlockSpec((NC, RB, D), lambda i: (0, i, 0)),
        pl.BlockSpec((NC, RB, HEADS), lambda i: (0, i, 0)),
        pl.BlockSpec((RB, HEADS), lambda i: (i, 0)),
        pl.BlockSpec((RB, HEADS), lambda i: (i, 0)),
        pl.BlockSpec((8, D), lambda i: (0, 0)),
        pl.BlockSpec((RB, D), lambda i: (i, 0)),
        pl.BlockSpec((RB, D), lambda i: (i, 0)),
        pl.BlockSpec((RB, 2), lambda i: (i, 0)),
        pl.BlockSpec((1, D), lambda i: (0, 0)),
        pl.BlockSpec((D, D), lambda i: (0, 0)),
    ],
    out_specs=[
        pl.BlockSpec((RB, D), lambda i: (i, 0)),
        pl.BlockSpec((RB, D), lambda i: (i, 0)),
    ],
    out_shape=[
        jax.ShapeDtypeStruct((N, D), jnp.float32),
        jax.ShapeDtypeStruct((N, D), jnp.float32),
    ],
)


def _tc4_body(agg_ref, h2_ref, dins_ref, bl_ref, out_ref):
    di = dins_ref[:, 0:1]
    s = dins_ref[:, 1:2]
    agg = agg_ref[0] + agg_ref[1]
    out_ref[...] = di * agg + s * h2_ref[...] + bl_ref[0, :]


_tc_final = pl.pallas_call(
    _tc4_body,
    grid=(GRID,),
    in_specs=[
        pl.BlockSpec((NC, RB, D), lambda i: (0, i, 0)),
        pl.BlockSpec((RB, D), lambda i: (i, 0)),
        pl.BlockSpec((RB, 2), lambda i: (i, 0)),
        pl.BlockSpec((1, D), lambda i: (0, 0)),
    ],
    out_specs=pl.BlockSpec((RB, D), lambda i: (i, 0)),
    out_shape=jax.ShapeDtypeStruct((N, D), jnp.float32),
)


def kernel(x, edge_index, edge_weight, W0, b0, Wg0, att_src0, att_dst0,
           bg0, Wl, bl):
    src = edge_index[0]
    dst = edge_index[1]
    b0r = b0.reshape(1, D)
    bgr = bg0.reshape(1, D)
    blr = bl.reshape(1, D)
    As = jnp.zeros((D, HEADS), jnp.float32)
    Ad = jnp.zeros((D, HEADS), jnp.float32)
    for h in range(HEADS):
        As = As.at[h * HC:(h + 1) * HC, h].set(att_src0[0, h, :])
        Ad = Ad.at[h * HC:(h + 1) * HC, h].set(att_dst0[0, h, :])

    degp = _sc_deg(dst, edge_weight).reshape(NC, N).transpose(1, 0)
    h0, h0d, dins = _tc_prep1(x, W0, degp)
    agg1p = _sc_gcn(h0d, src, dst, edge_weight)
    xp, h1, as4, ad4, gmax = _tc_prep2(x, agg1p, h0, dins, b0r, Wg0, As, Ad)
    agg2p, denp = _sc_gat(h1, src, dst, as4.reshape(-1), ad4.reshape(-1),
                          gmax.reshape(-1))
    den_t = denp.reshape(NC, HEADS, N).transpose(0, 2, 1)
    h2, h2d = _tc_epi2(agg2p, den_t, as4, ad4, gmax, h1, xp, dins, bgr, Wl)
    agg3p = _sc_gcn(h2d, src, dst, edge_weight)
    out = _tc_final(agg3p, h2, dins, blr)
    return out
